# baseline (speedup 1.0000x reference)
import functools

import jax
import jax.numpy as jnp
from jax import lax
from jax.experimental import pallas as pl
from jax.experimental.pallas import tpu as pltpu
from jax.experimental.pallas import tpu_sc as plsc

MARGIN_ = 1.0
NC, NS, L = 2, 16, 16
NW = NC * NS
U = 8
NEG_INF = float("-inf")
BIG = 1e30
SC_FRAC_NUM, SC_FRAC_DEN = 1, 4
SC_CHUNK = 16384
BLOCK_ROWS = 8192


def _worker_id():
    return lax.axis_index("s") * NC + lax.axis_index("c")


def _make_sc_stats(sc_base, n_sc, chunk):
    per_w = n_sc // NW
    n_chunks = per_w // chunk
    assert per_w % chunk == 0
    mesh = plsc.VectorSubcoreMesh(core_axis_name="c", subcore_axis_name="s")

    @functools.partial(
        pl.kernel,
        mesh=mesh,
        out_type=jax.ShapeDtypeStruct((NW, 3 * L), jnp.float32),
        scratch_types=[
            pltpu.VMEM((chunk,), jnp.float32),
            pltpu.VMEM((chunk,), jnp.float32),
            pltpu.VMEM((chunk,), jnp.int32),
            pltpu.VMEM((chunk,), jnp.int32),
            pltpu.VMEM((3 * L,), jnp.float32),
            pltpu.SemaphoreType.DMA,
            pltpu.SemaphoreType.DMA,
        ],
    )
    def sc_stats(scores_hbm, target_hbm, out_hbm,
                 sb0, sb1, tb0, tb1, rbuf, sem0, sem1):
        wid = _worker_id()
        base = sc_base + wid * per_w
        sbufs, tbufs, sems = (sb0, sb1), (tb0, tb1), (sem0, sem1)

        def issue(ci):
            b = ci % 2
            off = base + ci * chunk
            return (
                pltpu.async_copy(scores_hbm.at[pl.ds(off, chunk)], sbufs[b], sems[b]),
                pltpu.async_copy(target_hbm.at[pl.ds(off, chunk)], tbufs[b], sems[b]),
            )

        pend = [None, None]
        pend[0] = issue(0)
        carry = (
            tuple(jnp.full((L,), NEG_INF, jnp.float32) for _ in range(U)),
            tuple(jnp.zeros((L,), jnp.float32) for _ in range(U)),
            tuple(jnp.zeros((L,), jnp.int32) for _ in range(U)),
        )
        for ci in range(n_chunks):
            if ci + 1 < n_chunks:
                pend[(ci + 1) % 2] = issue(ci + 1)
            b = ci % 2
            d0, d1 = pend[b]
            d0.wait()
            d1.wait()
            sbuf, tbuf = sbufs[b], tbufs[b]

            def body(i, c, sbuf=sbuf, tbuf=tbuf):
                ms, ss, cs = list(c[0]), list(c[1]), list(c[2])
                for j in range(U):
                    s = sbuf[pl.ds(i + j * L, L)]
                    t = tbuf[pl.ds(i + j * L, L)]
                    neg = t == 0
                    ms[j] = jnp.maximum(ms[j], jnp.where(neg, s, NEG_INF))
                    ss[j] = ss[j] + jnp.where(neg, s, 0.0)
                    cs[j] = cs[j] + t
                return tuple(ms), tuple(ss), tuple(cs)

            carry = plsc.parallel_loop(0, chunk, step=U * L, carry=carry)(body)

        ms, ss, cs = carry
        m = functools.reduce(jnp.maximum, ms)
        sm = functools.reduce(jnp.add, ss)
        npos = functools.reduce(jnp.add, cs)
        nneg = ((per_w // L) - npos).astype(jnp.float32)
        rbuf[pl.ds(0, L)] = m
        rbuf[pl.ds(L, L)] = sm
        rbuf[pl.ds(2 * L, L)] = nneg
        pltpu.sync_copy(rbuf, out_hbm.at[wid])

    return sc_stats


def _make_tc_stats(n_tc, block_rows):
    rows = n_tc // 128
    assert rows % block_rows == 0
    grid = rows // block_rows

    def body(s_ref, t_ref, om_ref, oa_ref, op_ref, oc_ref):
        i = pl.program_id(0)
        x = s_ref[...]
        t = t_ref[...]
        tf = t.astype(jnp.float32)
        xm = jnp.max(x - tf * BIG, axis=0, keepdims=True)
        xa = jnp.sum(x, axis=0, keepdims=True)
        xp = jnp.sum(x * tf, axis=0, keepdims=True)
        xc = jnp.sum(t, axis=0, keepdims=True)

        @pl.when(i == 0)
        def _():
            om_ref[...] = xm
            oa_ref[...] = xa
            op_ref[...] = xp
            oc_ref[...] = xc

        @pl.when(i > 0)
        def _():
            om_ref[...] = jnp.maximum(om_ref[...], xm)
            oa_ref[...] = oa_ref[...] + xa
            op_ref[...] = op_ref[...] + xp
            oc_ref[...] = oc_ref[...] + xc

    return pl.pallas_call(
        body,
        grid=(grid,),
        in_specs=[
            pl.BlockSpec((block_rows, 128), lambda i: (i, 0)),
            pl.BlockSpec((block_rows, 128), lambda i: (i, 0)),
        ],
        out_specs=[pl.BlockSpec((1, 128), lambda i: (0, 0))] * 4,
        out_shape=[
            jax.ShapeDtypeStruct((1, 128), jnp.float32),
            jax.ShapeDtypeStruct((1, 128), jnp.float32),
            jax.ShapeDtypeStruct((1, 128), jnp.float32),
            jax.ShapeDtypeStruct((1, 128), jnp.int32),
        ],
    )


def _make_tc_combine(n, n_tc):
    def body(st_ref, tm_ref, ta_ref, tp_ref, tc_ref, out_ref):
        st = st_ref[...]
        lane = lax.broadcasted_iota(jnp.int32, (NW, 3 * L), 1)
        m_sc = jnp.max(jnp.where(lane < L, st, NEG_INF))
        sm_sc = jnp.sum(jnp.where((lane >= L) & (lane < 2 * L), st, 0.0))
        nn_sc = jnp.sum(jnp.where(lane >= 2 * L, st, 0.0))
        m_tc = jnp.max(tm_ref[...])
        np_tc = jnp.sum(tc_ref[...]).astype(jnp.float32)
        nn_tc = jnp.float32(n_tc) - np_tc
        sm_tc = jnp.sum(ta_ref[...]) - jnp.sum(tp_ref[...])
        c = MARGIN_ + jnp.maximum(m_sc, m_tc)
        n_neg = nn_sc + nn_tc
        sum_neg = sm_sc + sm_tc
        out_ref[0] = c
        out_ref[1] = jnp.where(n_neg > 0, c * n_neg - sum_neg, 0.0)
        out_ref[2] = jnp.float32(n) - n_neg

    return pl.pallas_call(
        body,
        out_specs=pl.BlockSpec(memory_space=pltpu.SMEM),
        out_shape=jax.ShapeDtypeStruct((4,), jnp.float32),
    )


def _make_tc_hinge(n, block_rows):
    rows = n // 128
    assert rows % block_rows == 0
    grid = rows // block_rows

    def body(c_ref, s_ref, out_ref, acc_ref):
        i = pl.program_id(0)

        @pl.when(i == 0)
        def _():
            acc_ref[...] = jnp.zeros_like(acc_ref)

        c = c_ref[0]
        acc_ref[...] = acc_ref[...] + jnp.sum(
            jnp.maximum(c - s_ref[...], 0.0), axis=0, keepdims=True)

        @pl.when(i == grid - 1)
        def _():
            total = jnp.sum(acc_ref[...])
            out_ref[0, 0] = (total - c_ref[1]) / c_ref[2]

    return pl.pallas_call(
        body,
        grid=(grid,),
        in_specs=[
            pl.BlockSpec(memory_space=pltpu.SMEM),
            pl.BlockSpec((block_rows, 128), lambda i: (i, 0)),
        ],
        out_specs=pl.BlockSpec(memory_space=pltpu.SMEM),
        out_shape=jax.ShapeDtypeStruct((1, 1), jnp.float32),
        scratch_shapes=[pltpu.VMEM((1, 128), jnp.float32)],
    )


def kernel(scores, target):
    n = scores.shape[0]
    n_sc = (n * SC_FRAC_NUM) // SC_FRAC_DEN
    n_tc = n - n_sc

    scores2d = scores.reshape(-1, 128)
    target2d = target.reshape(-1, 128)

    tm, ta, tp, tcnt = _make_tc_stats(n_tc, BLOCK_ROWS)(scores2d, target2d)
    st = _make_sc_stats(n_tc, n_sc, SC_CHUNK)(scores, target)

    cvec = _make_tc_combine(n, n_tc)(st, tm, ta, tp, tcnt)
    loss = _make_tc_hinge(n, BLOCK_ROWS)(cvec, scores2d)
    return loss.reshape(())


# speedup vs baseline: 1.0301x; 1.0301x over previous
import functools

import jax
import jax.numpy as jnp
from jax import lax
from jax.experimental import pallas as pl
from jax.experimental.pallas import tpu as pltpu
from jax.experimental.pallas import tpu_sc as plsc

MARGIN_ = 1.0
NC, NS, L = 2, 16, 16
NW = NC * NS
U = 8
NEG_INF = float("-inf")
BIG = 1e30
SC_FRAC_NUM, SC_FRAC_DEN = 1, 4
SC_CHUNK = 16384
BLOCK_ROWS = 8192


def _worker_id():
    return lax.axis_index("s") * NC + lax.axis_index("c")


def _make_sc_stats(sc_base, n_sc, chunk):
    per_w = n_sc // NW
    n_chunks = per_w // chunk
    assert per_w % chunk == 0
    mesh = plsc.VectorSubcoreMesh(core_axis_name="c", subcore_axis_name="s")

    @functools.partial(
        pl.kernel,
        mesh=mesh,
        out_type=jax.ShapeDtypeStruct((NW, 3 * L), jnp.float32),
        scratch_types=[
            pltpu.VMEM((chunk,), jnp.float32),
            pltpu.VMEM((chunk,), jnp.float32),
            pltpu.VMEM((chunk,), jnp.int32),
            pltpu.VMEM((chunk,), jnp.int32),
            pltpu.VMEM((3 * L,), jnp.float32),
            pltpu.SemaphoreType.DMA,
            pltpu.SemaphoreType.DMA,
        ],
    )
    def sc_stats(scores_hbm, target_hbm, out_hbm,
                 sb0, sb1, tb0, tb1, rbuf, sem0, sem1):
        wid = _worker_id()
        base = sc_base + wid * per_w
        sbufs, tbufs, sems = (sb0, sb1), (tb0, tb1), (sem0, sem1)

        def issue(ci):
            b = ci % 2
            off = base + ci * chunk
            return (
                pltpu.async_copy(scores_hbm.at[pl.ds(off, chunk)], sbufs[b], sems[b]),
                pltpu.async_copy(target_hbm.at[pl.ds(off, chunk)], tbufs[b], sems[b]),
            )

        pend = [None, None]
        pend[0] = issue(0)
        carry = (
            tuple(jnp.full((L,), NEG_INF, jnp.float32) for _ in range(U)),
            tuple(jnp.zeros((L,), jnp.float32) for _ in range(U)),
            tuple(jnp.zeros((L,), jnp.int32) for _ in range(U)),
        )
        for ci in range(n_chunks):
            if ci + 1 < n_chunks:
                pend[(ci + 1) % 2] = issue(ci + 1)
            b = ci % 2
            d0, d1 = pend[b]
            d0.wait()
            d1.wait()
            sbuf, tbuf = sbufs[b], tbufs[b]

            def body(i, c, sbuf=sbuf, tbuf=tbuf):
                ms, ss, cs = list(c[0]), list(c[1]), list(c[2])
                for j in range(U):
                    s = sbuf[pl.ds(i + j * L, L)]
                    t = tbuf[pl.ds(i + j * L, L)]
                    neg = t == 0
                    ms[j] = jnp.maximum(ms[j], jnp.where(neg, s, NEG_INF))
                    ss[j] = ss[j] + jnp.where(neg, s, 0.0)
                    cs[j] = cs[j] + t
                return tuple(ms), tuple(ss), tuple(cs)

            carry = plsc.parallel_loop(0, chunk, step=U * L, carry=carry)(body)

        ms, ss, cs = carry
        m = functools.reduce(jnp.maximum, ms)
        sm = functools.reduce(jnp.add, ss)
        npos = functools.reduce(jnp.add, cs)
        nneg = ((per_w // L) - npos).astype(jnp.float32)
        rbuf[pl.ds(0, L)] = m
        rbuf[pl.ds(L, L)] = sm
        rbuf[pl.ds(2 * L, L)] = nneg
        pltpu.sync_copy(rbuf, out_hbm.at[wid])

    return sc_stats


def _make_tc_stats(n_tc, block_rows):
    rows = n_tc // 128
    assert rows % block_rows == 0
    grid = rows // block_rows

    def body(s_ref, t_ref, om_ref, oa_ref, op_ref, oc_ref):
        i = pl.program_id(0)
        x = s_ref[...]
        t = t_ref[...]
        tf = t.astype(jnp.float32)
        xm = jnp.max(x - tf * BIG, axis=0, keepdims=True)
        xa = jnp.sum(x, axis=0, keepdims=True)
        xp = jnp.sum(x * tf, axis=0, keepdims=True)
        xc = jnp.sum(t, axis=0, keepdims=True)

        @pl.when(i == 0)
        def _():
            om_ref[...] = xm
            oa_ref[...] = xa
            op_ref[...] = xp
            oc_ref[...] = xc

        @pl.when(i > 0)
        def _():
            om_ref[...] = jnp.maximum(om_ref[...], xm)
            oa_ref[...] = oa_ref[...] + xa
            op_ref[...] = op_ref[...] + xp
            oc_ref[...] = oc_ref[...] + xc

    return pl.pallas_call(
        body,
        grid=(grid,),
        in_specs=[
            pl.BlockSpec((block_rows, 128), lambda i: (i, 0)),
            pl.BlockSpec((block_rows, 128), lambda i: (i, 0)),
        ],
        out_specs=[pl.BlockSpec((1, 128), lambda i: (0, 0))] * 4,
        out_shape=[
            jax.ShapeDtypeStruct((1, 128), jnp.float32),
            jax.ShapeDtypeStruct((1, 128), jnp.float32),
            jax.ShapeDtypeStruct((1, 128), jnp.float32),
            jax.ShapeDtypeStruct((1, 128), jnp.int32),
        ],
    )


def _make_tc_combine(n, n_tc):
    def body(st_ref, tm_ref, ta_ref, tp_ref, tc_ref, out_ref):
        st = st_ref[...]
        lane = lax.broadcasted_iota(jnp.int32, (NW, 3 * L), 1)
        m_sc = jnp.max(jnp.where(lane < L, st, NEG_INF))
        sm_sc = jnp.sum(jnp.where((lane >= L) & (lane < 2 * L), st, 0.0))
        nn_sc = jnp.sum(jnp.where(lane >= 2 * L, st, 0.0))
        m_tc = jnp.max(tm_ref[...])
        np_tc = jnp.sum(tc_ref[...]).astype(jnp.float32)
        nn_tc = jnp.float32(n_tc) - np_tc
        sm_tc = jnp.sum(ta_ref[...]) - jnp.sum(tp_ref[...])
        c = MARGIN_ + jnp.maximum(m_sc, m_tc)
        n_neg = nn_sc + nn_tc
        sum_neg = sm_sc + sm_tc
        out_ref[0] = c
        out_ref[1] = jnp.where(n_neg > 0, c * n_neg - sum_neg, 0.0)
        out_ref[2] = jnp.float32(n) - n_neg

    return pl.pallas_call(
        body,
        out_specs=pl.BlockSpec(memory_space=pltpu.SMEM),
        out_shape=jax.ShapeDtypeStruct((4,), jnp.float32),
    )


def _make_tc_hinge(n, block_rows):
    rows = n // 128
    half = rows // 2
    assert half % block_rows == 0
    grid = half // block_rows

    def body(c_ref, sa_ref, sb_ref, out_ref, acc_ref):
        i = pl.program_id(0)

        @pl.when(i == 0)
        def _():
            acc_ref[...] = jnp.zeros_like(acc_ref)

        c = c_ref[0]
        acc_ref[...] = (acc_ref[...]
                        + jnp.sum(jnp.maximum(c - sa_ref[...], 0.0),
                                  axis=0, keepdims=True)
                        + jnp.sum(jnp.maximum(c - sb_ref[...], 0.0),
                                  axis=0, keepdims=True))

        @pl.when(i == grid - 1)
        def _():
            total = jnp.sum(acc_ref[...])
            out_ref[0, 0] = (total - c_ref[1]) / c_ref[2]

    return pl.pallas_call(
        body,
        grid=(grid,),
        in_specs=[
            pl.BlockSpec(memory_space=pltpu.SMEM),
            pl.BlockSpec((block_rows, 128), lambda i: (i, 0)),
            pl.BlockSpec((block_rows, 128), lambda i: (i + grid, 0)),
        ],
        out_specs=pl.BlockSpec(memory_space=pltpu.SMEM),
        out_shape=jax.ShapeDtypeStruct((1, 1), jnp.float32),
        scratch_shapes=[pltpu.VMEM((1, 128), jnp.float32)],
    )


def kernel(scores, target):
    n = scores.shape[0]
    n_sc = (n * SC_FRAC_NUM) // SC_FRAC_DEN
    n_tc = n - n_sc

    scores2d = scores.reshape(-1, 128)
    target2d = target.reshape(-1, 128)

    tm, ta, tp, tcnt = _make_tc_stats(n_tc, BLOCK_ROWS)(scores2d, target2d)
    st = _make_sc_stats(n_tc, n_sc, SC_CHUNK)(scores, target)

    cvec = _make_tc_combine(n, n_tc)(st, tm, ta, tp, tcnt)
    loss = _make_tc_hinge(n, 4096)(cvec, scores2d, scores2d)
    return loss.reshape(())


# speedup vs baseline: 1.0315x; 1.0014x over previous
import functools

import jax
import jax.numpy as jnp
from jax import lax
from jax.experimental import pallas as pl
from jax.experimental.pallas import tpu as pltpu
from jax.experimental.pallas import tpu_sc as plsc

MARGIN_ = 1.0
NC, NS, L = 2, 16, 16
NW = NC * NS
U = 8
NEG_INF = float("-inf")
BIG = 1e30
SC_FRAC_NUM, SC_FRAC_DEN = 1, 4
SC_CHUNK = 16384
BLOCK_ROWS = 8192


def _worker_id():
    return lax.axis_index("s") * NC + lax.axis_index("c")


def _make_sc_stats(sc_base, n_sc, chunk):
    per_w = n_sc // NW
    n_chunks = per_w // chunk
    assert per_w % chunk == 0
    mesh = plsc.VectorSubcoreMesh(core_axis_name="c", subcore_axis_name="s")

    @functools.partial(
        pl.kernel,
        mesh=mesh,
        out_type=jax.ShapeDtypeStruct((NW, 3 * L), jnp.float32),
        scratch_types=[
            pltpu.VMEM((chunk,), jnp.float32),
            pltpu.VMEM((chunk,), jnp.float32),
            pltpu.VMEM((chunk,), jnp.int32),
            pltpu.VMEM((chunk,), jnp.int32),
            pltpu.VMEM((3 * L,), jnp.float32),
            pltpu.SemaphoreType.DMA,
            pltpu.SemaphoreType.DMA,
        ],
    )
    def sc_stats(scores_hbm, target_hbm, out_hbm,
                 sb0, sb1, tb0, tb1, rbuf, sem0, sem1):
        wid = _worker_id()
        base = sc_base + wid * per_w
        sbufs, tbufs, sems = (sb0, sb1), (tb0, tb1), (sem0, sem1)

        def issue(ci):
            b = ci % 2
            off = base + ci * chunk
            return (
                pltpu.async_copy(scores_hbm.at[pl.ds(off, chunk)], sbufs[b], sems[b]),
                pltpu.async_copy(target_hbm.at[pl.ds(off, chunk)], tbufs[b], sems[b]),
            )

        pend = [None, None]
        pend[0] = issue(0)
        carry = (
            tuple(jnp.full((L,), NEG_INF, jnp.float32) for _ in range(U)),
            tuple(jnp.zeros((L,), jnp.float32) for _ in range(U)),
            tuple(jnp.zeros((L,), jnp.int32) for _ in range(U)),
        )
        for ci in range(n_chunks):
            if ci + 1 < n_chunks:
                pend[(ci + 1) % 2] = issue(ci + 1)
            b = ci % 2
            d0, d1 = pend[b]
            d0.wait()
            d1.wait()
            sbuf, tbuf = sbufs[b], tbufs[b]

            def body(i, c, sbuf=sbuf, tbuf=tbuf):
                ms, ss, cs = list(c[0]), list(c[1]), list(c[2])
                for j in range(U):
                    s = sbuf[pl.ds(i + j * L, L)]
                    t = tbuf[pl.ds(i + j * L, L)]
                    neg = t == 0
                    ms[j] = jnp.maximum(ms[j], jnp.where(neg, s, NEG_INF))
                    ss[j] = ss[j] + jnp.where(neg, s, 0.0)
                    cs[j] = cs[j] + t
                return tuple(ms), tuple(ss), tuple(cs)

            carry = plsc.parallel_loop(0, chunk, step=U * L, carry=carry)(body)

        ms, ss, cs = carry
        m = functools.reduce(jnp.maximum, ms)
        sm = functools.reduce(jnp.add, ss)
        npos = functools.reduce(jnp.add, cs)
        nneg = ((per_w // L) - npos).astype(jnp.float32)
        rbuf[pl.ds(0, L)] = m
        rbuf[pl.ds(L, L)] = sm
        rbuf[pl.ds(2 * L, L)] = nneg
        pltpu.sync_copy(rbuf, out_hbm.at[wid])

    return sc_stats


def _make_tc_stats(n_tc, block_rows):
    rows = n_tc // 128
    half = rows // 2
    assert half % block_rows == 0
    grid = half // block_rows

    def body(sa_ref, sb_ref, ta_ref, tb_ref, om_ref, oa_ref, op_ref, oc_ref):
        i = pl.program_id(0)
        xm = None
        for s_ref, t_ref in ((sa_ref, ta_ref), (sb_ref, tb_ref)):
            x = s_ref[...]
            t = t_ref[...]
            tf = t.astype(jnp.float32)
            m = jnp.max(x - tf * BIG, axis=0, keepdims=True)
            a = jnp.sum(x, axis=0, keepdims=True)
            p = jnp.sum(x * tf, axis=0, keepdims=True)
            cc = jnp.sum(t, axis=0, keepdims=True)
            if xm is None:
                xm, xa, xp, xc = m, a, p, cc
            else:
                xm = jnp.maximum(xm, m)
                xa, xp, xc = xa + a, xp + p, xc + cc

        @pl.when(i == 0)
        def _():
            om_ref[...] = xm
            oa_ref[...] = xa
            op_ref[...] = xp
            oc_ref[...] = xc

        @pl.when(i > 0)
        def _():
            om_ref[...] = jnp.maximum(om_ref[...], xm)
            oa_ref[...] = oa_ref[...] + xa
            op_ref[...] = op_ref[...] + xp
            oc_ref[...] = oc_ref[...] + xc

    return pl.pallas_call(
        body,
        grid=(grid,),
        in_specs=[
            pl.BlockSpec((block_rows, 128), lambda i: (i, 0)),
            pl.BlockSpec((block_rows, 128), lambda i: (i + grid, 0)),
            pl.BlockSpec((block_rows, 128), lambda i: (i, 0)),
            pl.BlockSpec((block_rows, 128), lambda i: (i + grid, 0)),
        ],
        out_specs=[pl.BlockSpec((1, 128), lambda i: (0, 0))] * 4,
        out_shape=[
            jax.ShapeDtypeStruct((1, 128), jnp.float32),
            jax.ShapeDtypeStruct((1, 128), jnp.float32),
            jax.ShapeDtypeStruct((1, 128), jnp.float32),
            jax.ShapeDtypeStruct((1, 128), jnp.int32),
        ],
    )


def _make_tc_combine(n, n_tc):
    def body(st_ref, tm_ref, ta_ref, tp_ref, tc_ref, out_ref):
        st = st_ref[...]
        lane = lax.broadcasted_iota(jnp.int32, (NW, 3 * L), 1)
        m_sc = jnp.max(jnp.where(lane < L, st, NEG_INF))
        sm_sc = jnp.sum(jnp.where((lane >= L) & (lane < 2 * L), st, 0.0))
        nn_sc = jnp.sum(jnp.where(lane >= 2 * L, st, 0.0))
        m_tc = jnp.max(tm_ref[...])
        np_tc = jnp.sum(tc_ref[...]).astype(jnp.float32)
        nn_tc = jnp.float32(n_tc) - np_tc
        sm_tc = jnp.sum(ta_ref[...]) - jnp.sum(tp_ref[...])
        c = MARGIN_ + jnp.maximum(m_sc, m_tc)
        n_neg = nn_sc + nn_tc
        sum_neg = sm_sc + sm_tc
        out_ref[0] = c
        out_ref[1] = jnp.where(n_neg > 0, c * n_neg - sum_neg, 0.0)
        out_ref[2] = jnp.float32(n) - n_neg

    return pl.pallas_call(
        body,
        out_specs=pl.BlockSpec(memory_space=pltpu.SMEM),
        out_shape=jax.ShapeDtypeStruct((4,), jnp.float32),
    )


def _make_tc_hinge(n, block_rows):
    rows = n // 128
    half = rows // 2
    assert half % block_rows == 0
    grid = half // block_rows

    def body(c_ref, sa_ref, sb_ref, out_ref, acc_ref):
        i = pl.program_id(0)

        @pl.when(i == 0)
        def _():
            acc_ref[...] = jnp.zeros_like(acc_ref)

        c = c_ref[0]
        acc_ref[...] = (acc_ref[...]
                        + jnp.sum(jnp.maximum(c - sa_ref[...], 0.0),
                                  axis=0, keepdims=True)
                        + jnp.sum(jnp.maximum(c - sb_ref[...], 0.0),
                                  axis=0, keepdims=True))

        @pl.when(i == grid - 1)
        def _():
            total = jnp.sum(acc_ref[...])
            out_ref[0, 0] = (total - c_ref[1]) / c_ref[2]

    return pl.pallas_call(
        body,
        grid=(grid,),
        in_specs=[
            pl.BlockSpec(memory_space=pltpu.SMEM),
            pl.BlockSpec((block_rows, 128), lambda i: (i, 0)),
            pl.BlockSpec((block_rows, 128), lambda i: (i + grid, 0)),
        ],
        out_specs=pl.BlockSpec(memory_space=pltpu.SMEM),
        out_shape=jax.ShapeDtypeStruct((1, 1), jnp.float32),
        scratch_shapes=[pltpu.VMEM((1, 128), jnp.float32)],
    )


def kernel(scores, target):
    n = scores.shape[0]
    n_sc = (n * SC_FRAC_NUM) // SC_FRAC_DEN
    n_tc = n - n_sc

    scores2d = scores.reshape(-1, 128)
    target2d = target.reshape(-1, 128)

    tm, ta, tp, tcnt = _make_tc_stats(n_tc, 4096)(scores2d, scores2d,
                                                  target2d, target2d)
    st = _make_sc_stats(n_tc, n_sc, SC_CHUNK)(scores, target)

    cvec = _make_tc_combine(n, n_tc)(st, tm, ta, tp, tcnt)
    loss = _make_tc_hinge(n, 4096)(cvec, scores2d, scores2d)
    return loss.reshape(())


# speedup vs baseline: 1.0320x; 1.0004x over previous
import functools

import jax
import jax.numpy as jnp
from jax import lax
from jax.experimental import pallas as pl
from jax.experimental.pallas import tpu as pltpu
from jax.experimental.pallas import tpu_sc as plsc

MARGIN_ = 1.0
NC, NS, L = 2, 16, 16
NW = NC * NS
U = 8
NEG_INF = float("-inf")
BIG = 1e30
SC_FRAC_NUM, SC_FRAC_DEN = 1, 4
SC_CHUNK = 16384


def _worker_id():
    return lax.axis_index("s") * NC + lax.axis_index("c")


def _make_sc_stats(sc_base, n_sc, chunk):
    per_w = n_sc // NW
    n_chunks = per_w // chunk
    assert per_w % chunk == 0
    mesh = plsc.VectorSubcoreMesh(core_axis_name="c", subcore_axis_name="s")

    @functools.partial(
        pl.kernel,
        mesh=mesh,
        out_type=jax.ShapeDtypeStruct((NW, 3 * L), jnp.float32),
        scratch_types=[
            pltpu.VMEM((chunk,), jnp.float32),
            pltpu.VMEM((chunk,), jnp.float32),
            pltpu.VMEM((chunk,), jnp.int32),
            pltpu.VMEM((chunk,), jnp.int32),
            pltpu.VMEM((3 * L,), jnp.float32),
            pltpu.SemaphoreType.DMA,
            pltpu.SemaphoreType.DMA,
        ],
    )
    def sc_stats(scores_hbm, target_hbm, out_hbm,
                 sb0, sb1, tb0, tb1, rbuf, sem0, sem1):
        wid = _worker_id()
        base = sc_base + wid * per_w
        sbufs, tbufs, sems = (sb0, sb1), (tb0, tb1), (sem0, sem1)

        def issue(ci):
            b = ci % 2
            off = base + ci * chunk
            return (
                pltpu.async_copy(scores_hbm.at[pl.ds(off, chunk)], sbufs[b], sems[b]),
                pltpu.async_copy(target_hbm.at[pl.ds(off, chunk)], tbufs[b], sems[b]),
            )

        pend = [None, None]
        pend[0] = issue(0)
        carry = (
            tuple(jnp.full((L,), NEG_INF, jnp.float32) for _ in range(U)),
            tuple(jnp.zeros((L,), jnp.float32) for _ in range(U)),
            tuple(jnp.zeros((L,), jnp.int32) for _ in range(U)),
        )
        for ci in range(n_chunks):
            if ci + 1 < n_chunks:
                pend[(ci + 1) % 2] = issue(ci + 1)
            b = ci % 2
            d0, d1 = pend[b]
            d0.wait()
            d1.wait()
            sbuf, tbuf = sbufs[b], tbufs[b]

            def body(i, c, sbuf=sbuf, tbuf=tbuf):
                ms, ss, cs = list(c[0]), list(c[1]), list(c[2])
                for j in range(U):
                    s = sbuf[pl.ds(i + j * L, L)]
                    t = tbuf[pl.ds(i + j * L, L)]
                    neg = t == 0
                    ms[j] = jnp.maximum(ms[j], jnp.where(neg, s, NEG_INF))
                    ss[j] = ss[j] + jnp.where(neg, s, 0.0)
                    cs[j] = cs[j] + t
                return tuple(ms), tuple(ss), tuple(cs)

            carry = plsc.parallel_loop(0, chunk, step=U * L, carry=carry)(body)

        ms, ss, cs = carry
        m = functools.reduce(jnp.maximum, ms)
        sm = functools.reduce(jnp.add, ss)
        npos = functools.reduce(jnp.add, cs)
        nneg = ((per_w // L) - npos).astype(jnp.float32)
        rbuf[pl.ds(0, L)] = m
        rbuf[pl.ds(L, L)] = sm
        rbuf[pl.ds(2 * L, L)] = nneg
        pltpu.sync_copy(rbuf, out_hbm.at[wid])

    return sc_stats


def _make_tc_stats(n_tc, block_rows):
    rows = n_tc // 128
    half = rows // 2
    assert half % block_rows == 0
    grid = half // block_rows

    def body(sa_ref, sb_ref, ta_ref, tb_ref, om_ref, oa_ref, op_ref, oc_ref):
        i = pl.program_id(0)
        xm = None
        for s_ref, t_ref in ((sa_ref, ta_ref), (sb_ref, tb_ref)):
            x = s_ref[...]
            t = t_ref[...]
            tf = t.astype(jnp.float32)
            m = jnp.max(x - tf * BIG, axis=0, keepdims=True)
            a = jnp.sum(x, axis=0, keepdims=True)
            p = jnp.sum(x * tf, axis=0, keepdims=True)
            cc = jnp.sum(t, axis=0, keepdims=True)
            if xm is None:
                xm, xa, xp, xc = m, a, p, cc
            else:
                xm = jnp.maximum(xm, m)
                xa, xp, xc = xa + a, xp + p, xc + cc

        @pl.when(i == 0)
        def _():
            om_ref[...] = xm
            oa_ref[...] = xa
            op_ref[...] = xp
            oc_ref[...] = xc

        @pl.when(i > 0)
        def _():
            om_ref[...] = jnp.maximum(om_ref[...], xm)
            oa_ref[...] = oa_ref[...] + xa
            op_ref[...] = op_ref[...] + xp
            oc_ref[...] = oc_ref[...] + xc

    return pl.pallas_call(
        body,
        grid=(grid,),
        in_specs=[
            pl.BlockSpec((block_rows, 128), lambda i: (i, 0)),
            pl.BlockSpec((block_rows, 128), lambda i: (i + grid, 0)),
            pl.BlockSpec((block_rows, 128), lambda i: (i, 0)),
            pl.BlockSpec((block_rows, 128), lambda i: (i + grid, 0)),
        ],
        out_specs=[pl.BlockSpec((1, 128), lambda i: (0, 0))] * 4,
        out_shape=[
            jax.ShapeDtypeStruct((1, 128), jnp.float32),
            jax.ShapeDtypeStruct((1, 128), jnp.float32),
            jax.ShapeDtypeStruct((1, 128), jnp.float32),
            jax.ShapeDtypeStruct((1, 128), jnp.int32),
        ],
    )


def _make_tc_combine(n, n_tc):
    def body(st_ref, tm_ref, ta_ref, tp_ref, tc_ref, out_ref):
        st = st_ref[...]
        lane = lax.broadcasted_iota(jnp.int32, (NW, 3 * L), 1)
        m_sc = jnp.max(jnp.where(lane < L, st, NEG_INF))
        sm_sc = jnp.sum(jnp.where((lane >= L) & (lane < 2 * L), st, 0.0))
        nn_sc = jnp.sum(jnp.where(lane >= 2 * L, st, 0.0))
        m_tc = jnp.max(tm_ref[...])
        np_tc = jnp.sum(tc_ref[...]).astype(jnp.float32)
        nn_tc = jnp.float32(n_tc) - np_tc
        sm_tc = jnp.sum(ta_ref[...]) - jnp.sum(tp_ref[...])
        c = MARGIN_ + jnp.maximum(m_sc, m_tc)
        n_neg = nn_sc + nn_tc
        sum_neg = sm_sc + sm_tc
        out_ref[0] = c
        out_ref[1] = jnp.where(n_neg > 0, c * n_neg - sum_neg, 0.0)
        out_ref[2] = jnp.float32(n) - n_neg

    return pl.pallas_call(
        body,
        out_specs=pl.BlockSpec(memory_space=pltpu.SMEM),
        out_shape=jax.ShapeDtypeStruct((4,), jnp.float32),
    )


def _make_tc_hinge(n, block_rows):
    rows = n // 128
    half = rows // 2
    assert half % block_rows == 0
    grid = half // block_rows

    def body(c_ref, sa_ref, sb_ref, out_ref, acc_ref):
        i = pl.program_id(0)

        @pl.when(i == 0)
        def _():
            acc_ref[...] = jnp.zeros_like(acc_ref)

        c = c_ref[0]
        acc_ref[...] = (acc_ref[...]
                        + jnp.sum(jnp.maximum(c - sa_ref[...], 0.0),
                                  axis=0, keepdims=True)
                        + jnp.sum(jnp.maximum(c - sb_ref[...], 0.0),
                                  axis=0, keepdims=True))

        @pl.when(i == grid - 1)
        def _():
            total = jnp.sum(acc_ref[...])
            out_ref[0, 0] = (total - c_ref[1]) / c_ref[2]

    return pl.pallas_call(
        body,
        grid=(grid,),
        in_specs=[
            pl.BlockSpec(memory_space=pltpu.SMEM),
            pl.BlockSpec((block_rows, 128), lambda i: (i, 0)),
            pl.BlockSpec((block_rows, 128), lambda i: (i + grid, 0)),
        ],
        out_specs=pl.BlockSpec(memory_space=pltpu.SMEM),
        out_shape=jax.ShapeDtypeStruct((1, 1), jnp.float32),
        scratch_shapes=[pltpu.VMEM((1, 128), jnp.float32)],
    )


def kernel(scores, target):
    n = scores.shape[0]
    n_sc = (n * SC_FRAC_NUM) // SC_FRAC_DEN
    n_tc = n - n_sc

    scores2d = scores.reshape(-1, 128)
    target2d = target.reshape(-1, 128)

    tm, ta, tp, tcnt = _make_tc_stats(n_tc, 4096)(scores2d, scores2d,
                                                  target2d, target2d)
    st = _make_sc_stats(n_tc, n_sc, SC_CHUNK)(scores, target)

    cvec = _make_tc_combine(n, n_tc)(st, tm, ta, tp, tcnt)
    loss = _make_tc_hinge(n, 4096)(cvec, scores2d, scores2d)
    return loss.reshape(())
